# Initial kernel scaffold; baseline (speedup 1.0000x reference)
import jax
import jax.numpy as jnp
from jax.experimental import pallas as pl


def kernel(embedding, W1, b1, ln_gamma, ln_beta, W2, b2):
    raise NotImplementedError("write your pallas kernel here")


# speedup vs baseline: 1.4426x; 1.4426x over previous
import jax
import jax.numpy as jnp
from jax.experimental import pallas as pl
from jax.experimental.pallas import tpu as pltpu

_D = 256
_BLK = 2000


def _fused_mlp_kernel(x_ref, w1_ref, p_ref, o_ref):
    x = x_ref[...]
    h = jnp.dot(x, w1_ref[...], preferred_element_type=jnp.float32)
    h = h + p_ref[0:1, :]
    mu = jnp.mean(h, axis=1, keepdims=True)
    xc = h - mu
    var = jnp.mean(xc * xc, axis=1, keepdims=True)
    g = xc * jax.lax.rsqrt(var + 1e-5) * p_ref[1:2, :] + p_ref[2:3, :]
    g = jnp.maximum(g, 0.0)
    out = jnp.sum(g * p_ref[3:4, :], axis=1, keepdims=True) + p_ref[4, 0]
    o_ref[...] = out


def kernel(embedding, W1, b1, ln_gamma, ln_beta, W2, b2):
    B, N, D = embedding.shape
    M = B * N
    x = embedding.reshape(M, D)
    params = jnp.zeros((8, D), dtype=jnp.float32)
    params = params.at[0].set(b1)
    params = params.at[1].set(ln_gamma)
    params = params.at[2].set(ln_beta)
    params = params.at[3].set(W2[:, 0])
    params = params.at[4].set(jnp.full((D,), b2[0]))

    out = pl.pallas_call(
        _fused_mlp_kernel,
        grid=(M // _BLK,),
        in_specs=[
            pl.BlockSpec((_BLK, D), lambda i: (i, 0)),
            pl.BlockSpec((D, D), lambda i: (0, 0)),
            pl.BlockSpec((8, D), lambda i: (0, 0)),
        ],
        out_specs=pl.BlockSpec((_BLK, 1), lambda i: (i, 0)),
        out_shape=jax.ShapeDtypeStruct((M, 1), jnp.float32),
        compiler_params=pltpu.CompilerParams(
            dimension_semantics=("arbitrary",),
        ),
    )(x, W1, params)
    return out.reshape(B, N)
